# baseline (speedup 1.0000x reference)
import functools

import jax
import jax.numpy as jnp
from jax import lax
from jax.experimental import pallas as pl
from jax.experimental.pallas import tpu as pltpu

_BM = 400
_NBUF = 3
_NFULL = 24
_NOUT = _NFULL // _NBUF
_TB = 80
_NTAIL = 5


def _gcn_body(
    x_any, adj_any, w_ref, b_ref, o_any,
    xv, sup, ring, bstage, tstage,
    x_sem, ring_sems, tl_sems, st_sems, ts_sems,
):
    xcp = pltpu.make_async_copy(x_any, xv, x_sem)
    xcp.start()
    for k in range(_NBUF):
        pltpu.make_async_copy(
            adj_any.at[pl.ds(k * _BM, _BM), :], ring.at[k], ring_sems.at[k]
        ).start()
    xcp.wait()
    sup[...] = jnp.dot(xv[...], w_ref[...], preferred_element_type=jnp.float32)

    def step(i, carry):
        for s in range(_NBUF):
            c = i * _NBUF + s
            pltpu.make_async_copy(
                adj_any.at[pl.ds(c * _BM, _BM), :], ring.at[s], ring_sems.at[s]
            ).wait()

            @pl.when(c >= _NBUF)
            def _():
                pltpu.make_async_copy(
                    bstage.at[s],
                    o_any.at[pl.ds((c - _NBUF) * _BM, _BM), :],
                    st_sems.at[s],
                ).wait()

            bstage[s] = (
                jnp.dot(ring[s], sup[...], preferred_element_type=jnp.float32)
                + b_ref[...]
            )
            pltpu.make_async_copy(
                bstage.at[s], o_any.at[pl.ds(c * _BM, _BM), :], st_sems.at[s]
            ).start()

            @pl.when(c <= _NFULL - _NBUF - 1)
            def _():
                pltpu.make_async_copy(
                    adj_any.at[pl.ds((c + _NBUF) * _BM, _BM), :],
                    ring.at[s],
                    ring_sems.at[s],
                ).start()

            @pl.when(c == _NFULL - _NBUF)
            def _():
                for k in range(_NTAIL):
                    pltpu.make_async_copy(
                        adj_any.at[pl.ds(_NFULL * _BM + k * _TB, _TB), :],
                        ring.at[0, pl.ds(k * _TB, _TB)],
                        tl_sems.at[k],
                    ).start()

        return carry

    lax.fori_loop(0, _NOUT, step, 0)

    def tail_step(k, carry):
        pltpu.make_async_copy(
            adj_any.at[pl.ds(_NFULL * _BM + k * _TB, _TB), :],
            ring.at[0, pl.ds(k * _TB, _TB)],
            tl_sems.at[k],
        ).wait()
        tstage[pl.ds(k * _TB, _TB)] = (
            jnp.dot(
                ring[0, pl.ds(k * _TB, _TB)],
                sup[...],
                preferred_element_type=jnp.float32,
            )
            + b_ref[...]
        )
        pltpu.make_async_copy(
            tstage.at[pl.ds(k * _TB, _TB)],
            o_any.at[pl.ds(_NFULL * _BM + k * _TB, _TB), :],
            ts_sems.at[k],
        ).start()
        return carry

    lax.fori_loop(0, _NTAIL, tail_step, 0)

    for s in range(_NBUF):
        pltpu.make_async_copy(
            bstage.at[s],
            o_any.at[pl.ds((_NFULL - _NBUF + s) * _BM, _BM), :],
            st_sems.at[s],
        ).wait()

    def drain_step(k, carry):
        pltpu.make_async_copy(
            tstage.at[pl.ds(k * _TB, _TB)],
            o_any.at[pl.ds(_NFULL * _BM + k * _TB, _TB), :],
            ts_sems.at[k],
        ).wait()
        return carry

    lax.fori_loop(0, _NTAIL, drain_step, 0)


@jax.jit
def kernel(input, adj, W, b):
    n, d_in = input.shape
    d_out = W.shape[1]
    b2 = b.reshape(1, d_out)
    out = pl.pallas_call(
        _gcn_body,
        in_specs=[
            pl.BlockSpec(memory_space=pl.ANY),
            pl.BlockSpec(memory_space=pl.ANY),
            pl.BlockSpec((d_in, d_out), lambda: (0, 0)),
            pl.BlockSpec((1, d_out), lambda: (0, 0)),
        ],
        out_specs=pl.BlockSpec(memory_space=pl.ANY),
        out_shape=jax.ShapeDtypeStruct((n, d_out), jnp.float32),
        scratch_shapes=[
            pltpu.VMEM((n, d_in), jnp.float32),
            pltpu.VMEM((n, d_out), jnp.float32),
            pltpu.VMEM((_NBUF, _BM, n), jnp.float32),
            pltpu.VMEM((_NBUF, _BM, d_out), jnp.float32),
            pltpu.VMEM((_BM, d_out), jnp.float32),
            pltpu.SemaphoreType.DMA,
            pltpu.SemaphoreType.DMA((_NBUF,)),
            pltpu.SemaphoreType.DMA((_NTAIL,)),
            pltpu.SemaphoreType.DMA((_NBUF,)),
            pltpu.SemaphoreType.DMA((_NTAIL,)),
        ],
        compiler_params=pltpu.CompilerParams(
            vmem_limit_bytes=64 * 1024 * 1024,
        ),
    )(input, adj, W, b2)
    return out


# speedup vs baseline: 1.0178x; 1.0178x over previous
import functools

import jax
import jax.numpy as jnp
from jax.experimental import pallas as pl
from jax.experimental.pallas import tpu as pltpu

_BM = 544


def _gcn_body(x_ref, adj_ref, w_ref, b_ref, o_ref, sup_ref):
    @pl.when(pl.program_id(0) == 0)
    def _():
        sup_ref[...] = jnp.dot(
            x_ref[...], w_ref[...], preferred_element_type=jnp.float32
        )

    o_ref[...] = (
        jnp.dot(adj_ref[...], sup_ref[...], preferred_element_type=jnp.float32)
        + b_ref[...]
    )


@jax.jit
def kernel(input, adj, W, b):
    n, d_in = input.shape
    d_out = W.shape[1]
    b2 = b.reshape(1, d_out)
    grid = ((n + _BM - 1) // _BM,)
    out = pl.pallas_call(
        _gcn_body,
        grid=grid,
        in_specs=[
            pl.BlockSpec((n, d_in), lambda i: (0, 0)),
            pl.BlockSpec((_BM, n), lambda i: (i, 0)),
            pl.BlockSpec((d_in, d_out), lambda i: (0, 0)),
            pl.BlockSpec((1, d_out), lambda i: (0, 0)),
        ],
        out_specs=pl.BlockSpec((_BM, d_out), lambda i: (i, 0)),
        out_shape=jax.ShapeDtypeStruct((n, d_out), jnp.float32),
        scratch_shapes=[pltpu.VMEM((n, d_out), jnp.float32)],
        compiler_params=pltpu.CompilerParams(
            dimension_semantics=("arbitrary",),
            vmem_limit_bytes=64 * 1024 * 1024,
        ),
    )(input, adj, W, b2)
    return out


# speedup vs baseline: 1.0406x; 1.0224x over previous
import functools

import jax
import jax.numpy as jnp
from jax.experimental import pallas as pl
from jax.experimental.pallas import tpu as pltpu

_BM = 400


def _gcn_body(x_ref, adj_ref, w_ref, b_ref, o_ref, sup_ref):
    @pl.when(pl.program_id(0) == 0)
    def _():
        sup_ref[...] = jnp.dot(
            x_ref[...], w_ref[...], preferred_element_type=jnp.float32
        )

    o_ref[...] = (
        jnp.dot(adj_ref[...], sup_ref[...], preferred_element_type=jnp.float32)
        + b_ref[...]
    )


@jax.jit
def kernel(input, adj, W, b):
    n, d_in = input.shape
    d_out = W.shape[1]
    b2 = b.reshape(1, d_out)
    grid = (n // _BM,)
    out = pl.pallas_call(
        _gcn_body,
        grid=grid,
        in_specs=[
            pl.BlockSpec((n, d_in), lambda i: (0, 0)),
            pl.BlockSpec((_BM, n), lambda i: (i, 0)),
            pl.BlockSpec((d_in, d_out), lambda i: (0, 0)),
            pl.BlockSpec((1, d_out), lambda i: (0, 0)),
        ],
        out_specs=pl.BlockSpec((_BM, d_out), lambda i: (i, 0)),
        out_shape=jax.ShapeDtypeStruct((n, d_out), jnp.float32),
        scratch_shapes=[pltpu.VMEM((n, d_out), jnp.float32)],
        compiler_params=pltpu.CompilerParams(
            dimension_semantics=("arbitrary",),
            vmem_limit_bytes=64 * 1024 * 1024,
        ),
    )(input, adj, W, b2)
    return out


# speedup vs baseline: 1.0419x; 1.0013x over previous
import jax
import jax.numpy as jnp
from jax.experimental import pallas as pl
from jax.experimental.pallas import tpu as pltpu

_BM = 400


def _gcn_body(x_ref, adj_ref, w_ref, b_ref, o_ref, sup_ref):
    @pl.when(pl.program_id(0) == 0)
    def _():
        sup_ref[...] = jnp.dot(
            x_ref[...], w_ref[...], preferred_element_type=jnp.float32
        )

    o_ref[...] = (
        jnp.dot(adj_ref[...], sup_ref[...], preferred_element_type=jnp.float32)
        + b_ref[...]
    )


@jax.jit
def kernel(input, adj, W, b):
    n, d_in = input.shape
    d_out = W.shape[1]
    b2 = b.reshape(1, d_out)
    grid = (n // _BM,)
    out = pl.pallas_call(
        _gcn_body,
        grid=grid,
        in_specs=[
            pl.BlockSpec((n, d_in), lambda i: (0, 0)),
            pl.BlockSpec((_BM, n), lambda i: (i, 0)),
            pl.BlockSpec((d_in, d_out), lambda i: (0, 0)),
            pl.BlockSpec((1, d_out), lambda i: (0, 0)),
        ],
        out_specs=pl.BlockSpec((_BM, d_out), lambda i: (i, 0)),
        out_shape=jax.ShapeDtypeStruct((n, d_out), jnp.float32),
        scratch_shapes=[pltpu.VMEM((n, d_out), jnp.float32)],
        compiler_params=pltpu.CompilerParams(
            dimension_semantics=("arbitrary",),
            vmem_limit_bytes=64 * 1024 * 1024,
        ),
    )(input, adj, W, b2)
    return out
